# Initial kernel scaffold; baseline (speedup 1.0000x reference)
import jax
import jax.numpy as jnp
from jax import lax
from jax.experimental import pallas as pl
from jax.experimental.pallas import tpu as pltpu
from jax.experimental.pallas import tpu_sc as plsc

N = 10000
E = 320000
IN_CH = 18
HID = 256
HEADS = 8
C = 32
NHC = 4
HW = NHC * C
ROW = 144
K = 128
NS = 16
L = 16
RB = 80
NCH = N // RB
NECH = E // K
R = 1000
NB = N // R
EPS = 1e-16


def _sc_gat_body(tbl, srcg, dstg, attg, out,
                 src_v, dst_v, gl_i, gr_i, g_l, g_r, w_v, nbuf, att_v,
                 acc, sem1, sem2):
    c = lax.axis_index("c")
    s = lax.axis_index("s")

    pltpu.sync_copy(attg.at[c], att_v)

    zv = jnp.zeros((L,), jnp.float32)

    def zrow(r, carry):
        for j in range(ROW // L):
            nbuf[r, pl.ds(j * L, L)] = zv
        return carry

    lax.fori_loop(0, RB, zrow, 0)

    nz = (NCH - s + NS - 1) // NS

    def zchunk(t, carry):
        j = s + NS * t
        pltpu.sync_copy(nbuf, acc.at[pl.ds(j * RB, RB)])
        return carry

    lax.fori_loop(0, nz, zchunk, 0)

    plsc.subcore_barrier()

    att_vecs = [att_v[pl.ds(i * L, L)] for i in range(HW // L)]
    lane = lax.broadcasted_iota(jnp.int32, (L,), 0)

    off_l = c * N
    off_r = (2 + c) * N

    ne = (NECH - s + NS - 1) // NS

    def echunk(t, carry):
        j = s + NS * t
        base = j * K
        pltpu.sync_copy(srcg.at[pl.ds(base, K)], src_v)
        pltpu.sync_copy(dstg.at[pl.ds(base, K)], dst_v)

        def ib(i, cc):
            gl_i[pl.ds(i * L, L)] = src_v[pl.ds(i * L, L)] + off_l
            gr_i[pl.ds(i * L, L)] = dst_v[pl.ds(i * L, L)] + off_r
            return cc

        lax.fori_loop(0, K // L, ib, 0)

        cp1 = pltpu.async_copy(tbl.at[gl_i], g_l, sem1)
        cp2 = pltpu.async_copy(tbl.at[gr_i], g_r, sem2)
        cp1.wait()
        cp2.wait()

        def eb(e, cc):
            exrow = zv
            for h in range(NHC):
                gl0 = g_l[e, pl.ds(h * C, L)]
                gl1 = g_l[e, pl.ds(h * C + L, L)]
                gr0 = g_r[e, pl.ds(h * C, L)]
                gr1 = g_r[e, pl.ds(h * C + L, L)]
                t0 = gl0 + gr0
                t1 = gl1 + gr1
                t0 = jnp.maximum(t0, 0.2 * t0)
                t1 = jnp.maximum(t1, 0.2 * t1)
                sv = t0 * att_vecs[2 * h] + t1 * att_vecs[2 * h + 1]
                logit = jnp.sum(sv)
                ex = jnp.exp(jnp.full((L,), logit, jnp.float32))
                w_v[e, pl.ds(h * C, L)] = ex * gl0
                w_v[e, pl.ds(h * C + L, L)] = ex * gl1
                exrow = jnp.where(lane == h, ex, exrow)
            w_v[e, pl.ds(HW, L)] = exrow
            return cc

        lax.fori_loop(0, K, eb, 0)

        pltpu.sync_copy(w_v, acc.at[dst_v], add=True)
        return carry

    lax.fori_loop(0, ne, echunk, 0)

    plsc.subcore_barrier()

    def ochunk(t, carry):
        j = s + NS * t
        pltpu.sync_copy(acc.at[pl.ds(j * RB, RB)], nbuf)
        pltpu.sync_copy(nbuf, out.at[pl.ds(c * N + j * RB, RB)])
        return carry

    lax.fori_loop(0, nz, ochunk, 0)


def _sc_gat(tbl, src, dst, att_pair):
    f = pl.kernel(
        _sc_gat_body,
        out_type=jax.ShapeDtypeStruct((2 * N, ROW), jnp.float32),
        mesh=plsc.VectorSubcoreMesh(core_axis_name="c", subcore_axis_name="s"),
        scratch_types=[
            pltpu.VMEM((K,), jnp.int32),
            pltpu.VMEM((K,), jnp.int32),
            pltpu.VMEM((K,), jnp.int32),
            pltpu.VMEM((K,), jnp.int32),
            pltpu.VMEM((K, HW), jnp.float32),
            pltpu.VMEM((K, HW), jnp.float32),
            pltpu.VMEM((K, ROW), jnp.float32),
            pltpu.VMEM((RB, ROW), jnp.float32),
            pltpu.VMEM((HW,), jnp.float32),
            pltpu.VMEM_SHARED((N, ROW), jnp.float32),
            pltpu.SemaphoreType.DMA,
            pltpu.SemaphoreType.DMA,
        ],
    )
    return f(tbl, src, dst, att_pair)


def _mm1_block(x_ref, w_ref, b_ref, o_ref):
    o_ref[0] = (
        jnp.dot(x_ref[...], w_ref[0], preferred_element_type=jnp.float32)
        + b_ref[0]
    )


def _norm(xb, brd, bias_row):
    wsum = xb[:, :HW]
    den = xb[:, HW:HW + NHC]
    binv = jnp.dot(1.0 / (den + EPS), brd, preferred_element_type=jnp.float32)
    return jnp.maximum(wsum * binv + bias_row, 0.0)


def _mm2_block(x1_ref, x2_ref, w_ref, b_ref, brd_ref, bp_ref, o_ref):
    h_lo = _norm(x1_ref[...], brd_ref[...], bp_ref[0])
    h_hi = _norm(x2_ref[...], brd_ref[...], bp_ref[1])
    o_ref[0] = (
        jnp.dot(h_lo, w_ref[0, 0], preferred_element_type=jnp.float32)
        + jnp.dot(h_hi, w_ref[0, 1], preferred_element_type=jnp.float32)
        + b_ref[0]
    )


def _fin_block(x1_ref, x2_ref, x3_ref, x4_ref, f_ref, brd_ref, ba_ref,
               bf_ref, o_ref):
    xs = (x1_ref, x2_ref, x3_ref, x4_ref)
    acc = bf_ref[0]
    for p in range(4):
        h = _norm(xs[p][...], brd_ref[...], ba_ref[p])
        acc = acc + jnp.dot(h, f_ref[p], preferred_element_type=jnp.float32)
    o_ref[...] = acc


def _mm_layer1(x_pad, w, b):
    return pl.pallas_call(
        _mm1_block,
        grid=(NB, 4),
        in_specs=[
            pl.BlockSpec((R, HW), lambda i, q: (i, 0)),
            pl.BlockSpec((1, HW, HW), lambda i, q: (q, 0, 0)),
            pl.BlockSpec((1, HW), lambda i, q: (q, 0)),
        ],
        out_specs=pl.BlockSpec((1, R, HW), lambda i, q: (q, i, 0)),
        out_shape=jax.ShapeDtypeStruct((4, N, HW), jnp.float32),
    )(x_pad, w, b)


def _mm_layer2(acc1, w, b, brd, bias_pair):
    return pl.pallas_call(
        _mm2_block,
        grid=(NB, 4),
        in_specs=[
            pl.BlockSpec((R, ROW), lambda i, q: (i, 0)),
            pl.BlockSpec((R, ROW), lambda i, q: (NB + i, 0)),
            pl.BlockSpec((1, 2, HW, HW), lambda i, q: (q, 0, 0, 0)),
            pl.BlockSpec((1, HW), lambda i, q: (q, 0)),
            pl.BlockSpec((NHC, HW), lambda i, q: (0, 0)),
            pl.BlockSpec((2, HW), lambda i, q: (0, 0)),
        ],
        out_specs=pl.BlockSpec((1, R, HW), lambda i, q: (q, i, 0)),
        out_shape=jax.ShapeDtypeStruct((4, N, HW), jnp.float32),
    )(acc1, acc1, w, b, brd, bias_pair)


def _mm_final(acc1, acc2, f, brd, bias_all, bf_pad):
    return pl.pallas_call(
        _fin_block,
        grid=(NB,),
        in_specs=[
            pl.BlockSpec((R, ROW), lambda i: (i, 0)),
            pl.BlockSpec((R, ROW), lambda i: (NB + i, 0)),
            pl.BlockSpec((R, ROW), lambda i: (i, 0)),
            pl.BlockSpec((R, ROW), lambda i: (NB + i, 0)),
            pl.BlockSpec((4, HW, HW), lambda i: (0, 0, 0)),
            pl.BlockSpec((NHC, HW), lambda i: (0, 0)),
            pl.BlockSpec((4, HW), lambda i: (0, 0)),
            pl.BlockSpec((1, HW), lambda i: (0, 0)),
        ],
        out_specs=pl.BlockSpec((R, HW), lambda i: (i, 0)),
        out_shape=jax.ShapeDtypeStruct((N, HW), jnp.float32),
    )(acc1, acc1, acc2, acc2, f, brd, bias_all, bf_pad)


def kernel(x, edge_index, edge_attr, W1l, b1l, W1r, b1r, att1, bias1,
           W2l, b2l, W2r, b2r, att2, bias2, Wf, bf):
    del edge_attr

    src = edge_index[0]
    dst = edge_index[1]

    w1lT = jnp.pad(W1l.T, ((0, HW - IN_CH), (0, 0)))
    w1rT = jnp.pad(W1r.T, ((0, HW - IN_CH), (0, 0)))
    w1 = jnp.stack([w1lT[:, :HW], w1lT[:, HW:], w1rT[:, :HW], w1rT[:, HW:]])
    b1 = jnp.stack([b1l[:HW], b1l[HW:], b1r[:HW], b1r[HW:]])
    x_pad = jnp.pad(x, ((0, 0), (0, HW - IN_CH)))

    tbl1 = _mm_layer1(x_pad, w1, b1).reshape(4 * N, HW)
    acc1 = _sc_gat(tbl1, src, dst, att1.reshape(2, HW))

    w2lT = W2l.T
    w2rT = W2r.T
    w2 = jnp.stack([
        jnp.stack([w2lT[:HW, :HW], w2lT[HW:, :HW]]),
        jnp.stack([w2lT[:HW, HW:], w2lT[HW:, HW:]]),
        jnp.stack([w2rT[:HW, :HW], w2rT[HW:, :HW]]),
        jnp.stack([w2rT[:HW, HW:], w2rT[HW:, HW:]]),
    ])
    b2 = jnp.stack([b2l[:HW], b2l[HW:], b2r[:HW], b2r[HW:]])
    brd = (jnp.arange(HW)[None, :] // C == jnp.arange(NHC)[:, None]).astype(
        jnp.float32)
    bias_pair1 = bias1.reshape(2, HW)
    bias_pair2 = bias2.reshape(2, HW)

    tbl2 = _mm_layer2(acc1, w2, b2, brd, bias_pair1).reshape(4 * N, HW)
    acc2 = _sc_gat(tbl2, src, dst, att2.reshape(2, HW))

    fp = jnp.pad(Wf.T.reshape(4, HW, 4), ((0, 0), (0, 0), (0, HW - 4)))
    bias_all = jnp.concatenate([bias_pair1, bias_pair2])
    bf_pad = jnp.pad(bf, (0, HW - 4)).reshape(1, HW)

    ofull = _mm_final(acc1, acc2, fp, brd, bias_all, bf_pad)
    return ofull[:, :4]


# speedup vs baseline: 23.2251x; 23.2251x over previous
import jax
import jax.numpy as jnp
from jax import lax
from jax.experimental import pallas as pl
from jax.experimental.pallas import tpu as pltpu
from jax.experimental.pallas import tpu_sc as plsc

N = 10000
E = 320000
IN_CH = 18
HID = 256
HEADS = 8
C = 32
NHC = 4
HW = NHC * C
ND = N // 8
NDP = 1256
K = 80
NS = 16
L = 16
RB = 16
NECH = E // K
R = 1000
NB = N // R
EPS = 1e-16


def _sc_gat_body(tbl, srcg, dstg, attg, wout, dout,
                 src_v, dst_v, dpad, gl_i, gr_i, drow_i, g_l, g_r, nbuf,
                 att_v, acc, accd, sem1, sem2):
    c = lax.axis_index("c")
    s = lax.axis_index("s")

    pltpu.sync_copy(attg.at[c], att_v)

    zv = jnp.zeros((L,), jnp.float32)

    def zrow(r, carry):
        for j in range(HW // L):
            nbuf[r, pl.ds(j * L, L)] = zv
        return carry

    lax.fori_loop(0, RB, zrow, 0)

    nza = (N // RB - s + NS - 1) // NS

    def zchunk(t, carry):
        j = s + NS * t
        pltpu.sync_copy(nbuf, acc.at[pl.ds(j * RB, RB)])
        return carry

    lax.fori_loop(0, nza, zchunk, 0)

    nzd = (NDP // 8 - s + NS - 1) // NS

    def zdchunk(t, carry):
        j = s + NS * t
        pltpu.sync_copy(nbuf.at[pl.ds(0, 8)], accd.at[pl.ds(j * 8, 8)])
        return carry

    lax.fori_loop(0, nzd, zdchunk, 0)

    plsc.subcore_barrier()

    att_vecs = [att_v[pl.ds(i * L, L)] for i in range(HW // L)]
    lane = lax.broadcasted_iota(jnp.int32, (L,), 0)

    off_l = c * N
    off_r = (2 + c) * N

    ne = (NECH - s + NS - 1) // NS

    def echunk(t, carry):
        j = s + NS * t
        base = j * K
        pltpu.sync_copy(srcg.at[pl.ds(base, K)], src_v)
        pltpu.sync_copy(dstg.at[pl.ds(base, K)], dst_v)

        def ib(i, cc):
            dv = dst_v[pl.ds(i * L, L)]
            gl_i[pl.ds(i * L, L)] = src_v[pl.ds(i * L, L)] + off_l
            gr_i[pl.ds(i * L, L)] = dv + off_r
            drow_i[pl.ds(i * L, L)] = lax.shift_right_logical(dv, 3)
            dpad[pl.ds(i * L, L)] = dv
            return cc

        lax.fori_loop(0, K // L, ib, 0)

        cp1 = pltpu.async_copy(tbl.at[gl_i], g_l, sem1)
        cp2 = pltpu.async_copy(tbl.at[gr_i], g_r, sem2)
        cp1.wait()
        cp2.wait()

        def eb(e, cc):
            exrow = zv
            gls = []
            exs = []
            for h in range(NHC):
                gl0 = g_l[e, pl.ds(h * C, L)]
                gl1 = g_l[e, pl.ds(h * C + L, L)]
                gr0 = g_r[e, pl.ds(h * C, L)]
                gr1 = g_r[e, pl.ds(h * C + L, L)]
                t0 = gl0 + gr0
                t1 = gl1 + gr1
                t0 = jnp.maximum(t0, 0.2 * t0)
                t1 = jnp.maximum(t1, 0.2 * t1)
                sv = t0 * att_vecs[2 * h] + t1 * att_vecs[2 * h + 1]
                logit = jnp.sum(sv)
                ex = jnp.exp(jnp.full((L,), logit, jnp.float32))
                gls.append((gl0, gl1))
                exs.append(ex)
                exrow = jnp.where(lane == h, ex, exrow)
            for h in range(NHC):
                g_l[e, pl.ds(h * C, L)] = exs[h] * gls[h][0]
                g_l[e, pl.ds(h * C + L, L)] = exs[h] * gls[h][1]
            m = dpad[pl.ds(e, L)][0] & 7
            for mm in range(8):
                g_r[e, pl.ds(mm * L, L)] = jnp.where(m == mm, exrow, zv)
            return cc

        lax.fori_loop(0, K, eb, 0)

        pltpu.sync_copy(g_l, acc.at[dst_v], add=True)
        pltpu.sync_copy(g_r, accd.at[drow_i], add=True)
        return carry

    lax.fori_loop(0, ne, echunk, 0)

    plsc.subcore_barrier()

    def ochunk(t, carry):
        j = s + NS * t
        pltpu.sync_copy(acc.at[pl.ds(j * RB, RB)], nbuf)
        pltpu.sync_copy(nbuf, wout.at[pl.ds(c * N + j * RB, RB)])
        return carry

    lax.fori_loop(0, nza, ochunk, 0)

    def odchunk(t, carry):
        j = s + NS * t
        pltpu.sync_copy(accd.at[pl.ds(j * 8, 8)], nbuf.at[pl.ds(0, 8)])
        pltpu.sync_copy(nbuf.at[pl.ds(0, 8)], dout.at[c, pl.ds(j * 8, 8)])
        return carry

    lax.fori_loop(0, nzd, odchunk, 0)


def _sc_gat(tbl, src, dst, att_pair):
    f = pl.kernel(
        _sc_gat_body,
        out_type=(
            jax.ShapeDtypeStruct((2 * N, HW), jnp.float32),
            jax.ShapeDtypeStruct((2, NDP, HW), jnp.float32),
        ),
        mesh=plsc.VectorSubcoreMesh(core_axis_name="c", subcore_axis_name="s"),
        compiler_params=pltpu.CompilerParams(needs_layout_passes=False),
        scratch_types=[
            pltpu.VMEM((K,), jnp.int32),
            pltpu.VMEM((K,), jnp.int32),
            pltpu.VMEM((K + L,), jnp.int32),
            pltpu.VMEM((K,), jnp.int32),
            pltpu.VMEM((K,), jnp.int32),
            pltpu.VMEM((K,), jnp.int32),
            pltpu.VMEM((K, HW), jnp.float32),
            pltpu.VMEM((K, HW), jnp.float32),
            pltpu.VMEM((RB, HW), jnp.float32),
            pltpu.VMEM((HW,), jnp.float32),
            pltpu.VMEM_SHARED((N, HW), jnp.float32),
            pltpu.VMEM_SHARED((NDP, HW), jnp.float32),
            pltpu.SemaphoreType.DMA,
            pltpu.SemaphoreType.DMA,
        ],
    )
    return f(tbl, src, dst, att_pair)


def _mm1_block(x_ref, w_ref, b_ref, o_ref):
    o_ref[0] = (
        jnp.dot(x_ref[...], w_ref[0], preferred_element_type=jnp.float32)
        + b_ref[0, 0]
    )


def _norm(xb, d_ref, brd, bias_row):
    binv = jnp.dot(1.0 / (d_ref[0] + EPS), brd,
                   preferred_element_type=jnp.float32)
    return jnp.maximum(xb * binv + bias_row, 0.0)


def _mm2_block(x1_ref, x2_ref, d1_ref, d2_ref, w_ref, b_ref, brd_ref, bp_ref,
               o_ref):
    h_lo = _norm(x1_ref[...], d1_ref, brd_ref[...], bp_ref[0])
    h_hi = _norm(x2_ref[...], d2_ref, brd_ref[...], bp_ref[1])
    o_ref[0] = (
        jnp.dot(h_lo, w_ref[0, 0], preferred_element_type=jnp.float32)
        + jnp.dot(h_hi, w_ref[0, 1], preferred_element_type=jnp.float32)
        + b_ref[0, 0]
    )


def _fin_block(x1_ref, x2_ref, x3_ref, x4_ref, d1_ref, d2_ref, d3_ref, d4_ref,
               f_ref, brd_ref, ba_ref, bf_ref, o_ref):
    xs = (x1_ref, x2_ref, x3_ref, x4_ref)
    ds_ = (d1_ref, d2_ref, d3_ref, d4_ref)
    acc = bf_ref[0]
    for p in range(4):
        h = _norm(xs[p][...], ds_[p], brd_ref[...], ba_ref[p])
        acc = acc + jnp.dot(h, f_ref[p], preferred_element_type=jnp.float32)
    o_ref[...] = acc


def _mm_layer1(x_pad, w, b):
    return pl.pallas_call(
        _mm1_block,
        grid=(NB, 4),
        in_specs=[
            pl.BlockSpec((R, HW), lambda i, q: (i, 0)),
            pl.BlockSpec((1, HW, HW), lambda i, q: (q, 0, 0)),
            pl.BlockSpec((1, 1, HW), lambda i, q: (q, 0, 0)),
        ],
        out_specs=pl.BlockSpec((1, R, HW), lambda i, q: (q, i, 0)),
        out_shape=jax.ShapeDtypeStruct((4, N, HW), jnp.float32),
    )(x_pad, w, b)


def _mm_layer2(wout1, den1, w, b, brd, bias_pair):
    return pl.pallas_call(
        _mm2_block,
        grid=(NB, 4),
        in_specs=[
            pl.BlockSpec((R, HW), lambda i, q: (i, 0)),
            pl.BlockSpec((R, HW), lambda i, q: (NB + i, 0)),
            pl.BlockSpec((1, R, NHC), lambda i, q: (0, i, 0)),
            pl.BlockSpec((1, R, NHC), lambda i, q: (1, i, 0)),
            pl.BlockSpec((1, 2, HW, HW), lambda i, q: (q, 0, 0, 0)),
            pl.BlockSpec((1, 1, HW), lambda i, q: (q, 0, 0)),
            pl.BlockSpec((NHC, HW), lambda i, q: (0, 0)),
            pl.BlockSpec((2, HW), lambda i, q: (0, 0)),
        ],
        out_specs=pl.BlockSpec((1, R, HW), lambda i, q: (q, i, 0)),
        out_shape=jax.ShapeDtypeStruct((4, N, HW), jnp.float32),
    )(wout1, wout1, den1, den1, w, b, brd, bias_pair)


def _mm_final(wout1, den1, wout2, den2, f, brd, bias_all, bf_pad):
    return pl.pallas_call(
        _fin_block,
        grid=(NB,),
        in_specs=[
            pl.BlockSpec((R, HW), lambda i: (i, 0)),
            pl.BlockSpec((R, HW), lambda i: (NB + i, 0)),
            pl.BlockSpec((R, HW), lambda i: (i, 0)),
            pl.BlockSpec((R, HW), lambda i: (NB + i, 0)),
            pl.BlockSpec((1, R, NHC), lambda i: (0, i, 0)),
            pl.BlockSpec((1, R, NHC), lambda i: (1, i, 0)),
            pl.BlockSpec((1, R, NHC), lambda i: (0, i, 0)),
            pl.BlockSpec((1, R, NHC), lambda i: (1, i, 0)),
            pl.BlockSpec((4, HW, HW), lambda i: (0, 0, 0)),
            pl.BlockSpec((NHC, HW), lambda i: (0, 0)),
            pl.BlockSpec((4, HW), lambda i: (0, 0)),
            pl.BlockSpec((1, HW), lambda i: (0, 0)),
        ],
        out_specs=pl.BlockSpec((R, HW), lambda i: (i, 0)),
        out_shape=jax.ShapeDtypeStruct((N, HW), jnp.float32),
    )(wout1, wout1, wout2, wout2, den1, den1, den2, den2,
      f, brd, bias_all, bf_pad)


def kernel(x, edge_index, edge_attr, W1l, b1l, W1r, b1r, att1, bias1,
           W2l, b2l, W2r, b2r, att2, bias2, Wf, bf):
    del edge_attr

    src = edge_index[0]
    dst = edge_index[1]

    w1lT = jnp.pad(W1l.T, ((0, HW - IN_CH), (0, 0)))
    w1rT = jnp.pad(W1r.T, ((0, HW - IN_CH), (0, 0)))
    w1 = jnp.stack([w1lT[:, :HW], w1lT[:, HW:], w1rT[:, :HW], w1rT[:, HW:]])
    b1 = jnp.stack([b1l[:HW], b1l[HW:], b1r[:HW], b1r[HW:]]).reshape(4, 1, HW)
    x_pad = jnp.pad(x, ((0, 0), (0, HW - IN_CH)))

    tbl1 = _mm_layer1(x_pad, w1, b1).reshape(4 * N, HW)
    wout1, dout1 = _sc_gat(tbl1, src, dst, att1.reshape(2, HW))
    den1 = dout1[:, :ND].reshape(2, ND, 8, L)[:, :, :, :NHC].reshape(2, N, NHC)

    w2lT = W2l.T
    w2rT = W2r.T
    w2 = jnp.stack([
        jnp.stack([w2lT[:HW, :HW], w2lT[HW:, :HW]]),
        jnp.stack([w2lT[:HW, HW:], w2lT[HW:, HW:]]),
        jnp.stack([w2rT[:HW, :HW], w2rT[HW:, :HW]]),
        jnp.stack([w2rT[:HW, HW:], w2rT[HW:, HW:]]),
    ])
    b2 = jnp.stack([b2l[:HW], b2l[HW:], b2r[:HW], b2r[HW:]]).reshape(4, 1, HW)
    brd = (jnp.arange(HW)[None, :] // C == jnp.arange(NHC)[:, None]).astype(
        jnp.float32)
    bias_pair1 = bias1.reshape(2, HW)
    bias_pair2 = bias2.reshape(2, HW)

    tbl2 = _mm_layer2(wout1, den1, w2, b2, brd, bias_pair1).reshape(4 * N, HW)
    wout2, dout2 = _sc_gat(tbl2, src, dst, att2.reshape(2, HW))
    den2 = dout2[:, :ND].reshape(2, ND, 8, L)[:, :, :, :NHC].reshape(2, N, NHC)

    fp = jnp.pad(Wf.T.reshape(4, HW, 4), ((0, 0), (0, 0), (0, HW - 4)))
    bias_all = jnp.concatenate([bias_pair1, bias_pair2])
    bf_pad = jnp.pad(bf, (0, HW - 4)).reshape(1, HW)

    ofull = _mm_final(wout1, den1, wout2, den2, fp, brd, bias_all, bf_pad)
    return ofull[:, :4]
